# baseline (speedup 1.0000x reference)
import functools

import jax
import jax.numpy as jnp
import numpy as np
from jax import lax
from jax.experimental import pallas as pl
from jax.experimental.pallas import tpu as pltpu
from jax.experimental.pallas import tpu_sc as plsc

N = 10000
E = 320000
D = 128
H = 4
DH = D // H
COEF = 1.0 / np.sqrt(DH)

NC = 2
NS = 16
HL = H // NC
HD = HL * DH
EPT = E // NS
CHUNK = 80
SUP = 4000
NSUP = EPT // SUP
NPAIR = SUP // (2 * CHUNK)
NPAD = 10240
NP2 = NPAD // 2
RPT = NP2 // NS


def _qkv_body(x_ref, wq_ref, bq_ref, wkv_ref, bkv_ref, q_ref, kv_ref):
    xb = x_ref[...]
    q_ref[...] = jnp.dot(xb, wq_ref[...], preferred_element_type=jnp.float32) + bq_ref[...]
    for c in range(NC):
        kv_ref[c] = jnp.dot(xb, wkv_ref[c], preferred_element_type=jnp.float32) + bkv_ref[c]


def _qkv_call(x, wq, bq, wkv, bkv):
    blk = 1000
    nb = N // blk
    return pl.pallas_call(
        _qkv_body,
        grid=(nb,),
        in_specs=[
            pl.BlockSpec((blk, D), lambda j: (j, 0)),
            pl.BlockSpec((D, D), lambda j: (0, 0)),
            pl.BlockSpec((1, D), lambda j: (0, 0)),
            pl.BlockSpec((NC, D, 2 * HD), lambda j: (0, 0, 0)),
            pl.BlockSpec((NC, 1, 2 * HD), lambda j: (0, 0, 0)),
        ],
        out_specs=[
            pl.BlockSpec((blk, D), lambda j: (j, 0)),
            pl.BlockSpec((NC, blk, 2 * HD), lambda j: (0, j, 0)),
        ],
        out_shape=[
            jax.ShapeDtypeStruct((N, D), jnp.float32),
            jax.ShapeDtypeStruct((NC, N, 2 * HD), jnp.float32),
        ],
    )(x, wq, bq, wkv, bkv)


def _sc_edge_body(q_hbm, kv_hbm, src_hbm, dst_hbm, acc_hbm, z_hbm,
                  src_all, dst_all,
                  gsrcA, ddstA, ridxA, gsrcB, ddstB, ridxB,
                  qbufA, kvbufA, qbufB, kvbufB, obuf, zpriv, shared,
                  semA0, semA1, semB0, semB1):
    c = lax.axis_index("c")
    s = lax.axis_index("s")
    wid = c * NS + s

    lanes = lax.iota(jnp.int32, 16)
    zeros16 = jnp.zeros((16,), jnp.float32)
    ones16 = jnp.ones((16,), jnp.float32)
    mlow = lanes < HL
    zoff = jnp.where(mlow, lanes * NPAD, 0)
    cnv = jnp.broadcast_to(c * N, (16,)).astype(jnp.int32)
    rot8 = (((lanes + 8) % 16)[:, None])
    perms8 = [(((lanes & 8) | ((lanes + sh) & 7))[:, None]) for sh in (4, 2, 1)]
    splat0 = jnp.zeros((16, 1), jnp.int32)
    splat8 = jnp.full((16, 1), 8, jnp.int32)
    perm08 = jnp.where(lanes == 1, 8, 0)[:, None]
    mask8 = lanes < 8
    dnums = lax.GatherDimensionNumbers(
        offset_dims=(), collapsed_slice_dims=(0,), start_index_map=(0,))

    def _g16(v, p):
        return lax.gather(v, p, dnums, slice_sizes=(1,),
                          mode=lax.GatherScatterMode.PROMISE_IN_BOUNDS)

    def _zero_obuf(i, _):
        for j in range(D // 16):
            obuf[i, pl.ds(16 * j, 16)] = zeros16
        return 0

    def _zero_zpriv(i, _):
        zpriv[pl.ds(16 * i, 16)] = zeros16
        return 0

    lax.fori_loop(0, CHUNK, _zero_obuf, 0)
    lax.fori_loop(0, HL * NPAD // 16, _zero_zpriv, 0)
    for rep in range(RPT // CHUNK):
        pltpu.sync_copy(obuf, shared.at[pl.ds(s * RPT + rep * CHUNK, CHUNK)])
    plsc.subcore_barrier()

    def _fill(j, gsrc, ddst, ridx):
        def _mk(b, _):
            sv = src_all[pl.ds(j * CHUNK + 16 * b, 16)]
            dv = dst_all[pl.ds(j * CHUNK + 16 * b, 16)]
            gsrc[pl.ds(16 * b, 16)] = sv + cnv
            ddst[pl.ds(16 * b, 16)] = dv
            ridx[pl.ds(16 * b, 16)] = lax.shift_right_logical(dv, 1)
            return 0
        lax.fori_loop(0, CHUNK // 16, _mk, 0)

    def _issue(gsrc, ddst, qb, kvb, sem0, sem1):
        pltpu.async_copy(kv_hbm.at[gsrc], kvb, sem0)
        pltpu.async_copy(q_hbm.at[ddst], qb, sem1)

    def _wait(gsrc, ddst, qb, kvb, sem0, sem1):
        pltpu.make_async_copy(kv_hbm.at[gsrc], kvb, sem0).wait()
        pltpu.make_async_copy(q_hbm.at[ddst], qb, sem1).wait()

    def _compute(qb, kvb, ddst, ridx):
        def _grp(b, _):
            dv = ddst[pl.ds(16 * b, 16)]
            for j in range(16):
                e = 16 * b + j
                dsp = _g16(dv, jnp.full((16, 1), j, jnp.int32))
                m1 = (dsp & 1).astype(jnp.float32)
                m0 = ones16 - m1
                qv = [qb[e, pl.ds(c * HD + 16 * t, 16)] for t in range(HL * 2)]
                kk = [kvb[e, pl.ds(16 * t, 16)] for t in range(HL * 2)]
                vv = [kvb[e, pl.ds(HD + 16 * t, 16)] for t in range(HL * 2)]
                part0 = kk[0] * qv[0] + kk[1] * qv[1]
                part1 = kk[2] * qv[2] + kk[3] * qv[3]
                a0 = part0 + _g16(part0, rot8)
                a1 = part1 + _g16(part1, rot8)
                u = jnp.where(mask8, a0, a1)
                for pp in perms8:
                    u = u + _g16(u, pp)
                exu = jnp.exp(u)
                ex0 = _g16(exu, splat0)
                ex1 = _g16(exu, splat8)
                v00 = vv[0] * ex0
                v01 = vv[1] * ex0
                v10 = vv[2] * ex1
                v11 = vv[3] * ex1
                obuf[e, pl.ds(0, 16)] = v00 * m0
                obuf[e, pl.ds(16, 16)] = v01 * m0
                obuf[e, pl.ds(32, 16)] = v10 * m0
                obuf[e, pl.ds(48, 16)] = v11 * m0
                obuf[e, pl.ds(HD, 16)] = v00 * m1
                obuf[e, pl.ds(HD + 16, 16)] = v01 * m1
                obuf[e, pl.ds(HD + 32, 16)] = v10 * m1
                obuf[e, pl.ds(HD + 48, 16)] = v11 * m1
                zvec = _g16(exu, perm08)
                plsc.addupdate_scatter(zpriv, [zoff + dsp], zvec, mask=mlow)
            return 0

        lax.fori_loop(0, CHUNK // 16, _grp, 0)
        pltpu.sync_copy(obuf, shared.at[ridx], add=True)

    def _super(u, _):
        pltpu.sync_copy(src_hbm.at[pl.ds(s * EPT + u * SUP, SUP)], src_all)
        pltpu.sync_copy(dst_hbm.at[pl.ds(s * EPT + u * SUP, SUP)], dst_all)
        _fill(jnp.int32(0), gsrcA, ddstA, ridxA)
        _issue(gsrcA, ddstA, qbufA, kvbufA, semA0, semA1)

        def _pair(i, _):
            _fill(2 * i + 1, gsrcB, ddstB, ridxB)
            _issue(gsrcB, ddstB, qbufB, kvbufB, semB0, semB1)
            _wait(gsrcA, ddstA, qbufA, kvbufA, semA0, semA1)
            _compute(qbufA, kvbufA, ddstA, ridxA)

            @pl.when(i < NPAIR - 1)
            def _prefetch_next():
                _fill(2 * i + 2, gsrcA, ddstA, ridxA)
                _issue(gsrcA, ddstA, qbufA, kvbufA, semA0, semA1)

            _wait(gsrcB, ddstB, qbufB, kvbufB, semB0, semB1)
            _compute(qbufB, kvbufB, ddstB, ridxB)
            return 0

        lax.fori_loop(0, NPAIR, _pair, 0)
        return 0

    lax.fori_loop(0, NSUP, _super, 0)

    plsc.subcore_barrier()
    pltpu.sync_copy(shared.at[pl.ds(s * RPT, RPT)],
                    acc_hbm.at[c, pl.ds(s * RPT, RPT)])
    pltpu.sync_copy(zpriv, z_hbm.at[wid])


def _sc_edge_call(q, kv, src, dst):
    mesh = plsc.VectorSubcoreMesh(core_axis_name="c", subcore_axis_name="s")
    kern = functools.partial(
        pl.kernel,
        mesh=mesh,
        compiler_params=pltpu.CompilerParams(needs_layout_passes=False),
        out_type=[
            jax.ShapeDtypeStruct((NC, NP2, D), jnp.float32),
            jax.ShapeDtypeStruct((NC * NS, HL * NPAD), jnp.float32),
        ],
        scratch_types=[
            pltpu.VMEM((SUP,), jnp.int32),
            pltpu.VMEM((SUP,), jnp.int32),
            pltpu.VMEM((CHUNK,), jnp.int32),
            pltpu.VMEM((CHUNK,), jnp.int32),
            pltpu.VMEM((CHUNK,), jnp.int32),
            pltpu.VMEM((CHUNK,), jnp.int32),
            pltpu.VMEM((CHUNK,), jnp.int32),
            pltpu.VMEM((CHUNK,), jnp.int32),
            pltpu.VMEM((CHUNK, D), jnp.float32),
            pltpu.VMEM((CHUNK, D), jnp.float32),
            pltpu.VMEM((CHUNK, D), jnp.float32),
            pltpu.VMEM((CHUNK, D), jnp.float32),
            pltpu.VMEM((CHUNK, D), jnp.float32),
            pltpu.VMEM((HL * NPAD,), jnp.float32),
            pltpu.VMEM_SHARED((NP2, D), jnp.float32),
            pltpu.SemaphoreType.DMA,
            pltpu.SemaphoreType.DMA,
            pltpu.SemaphoreType.DMA,
            pltpu.SemaphoreType.DMA,
        ],
    )(_sc_edge_body)
    return kern(q, kv, src, dst)


def _out_body(x_ref, a0_ref, a1_ref, z_ref, w1_ref, w2a_ref, w2b_ref, b_ref,
              o_ref):
    blk = o_ref.shape[0]
    half = blk // 2
    rows = lax.broadcasted_iota(jnp.int32, (blk, half), 0)
    cols = lax.broadcasted_iota(jnp.int32, (blk, half), 1)
    s0 = jnp.where(rows == 2 * cols, 1.0, 0.0).astype(jnp.float32)
    s1 = jnp.where(rows == 2 * cols + 1, 1.0, 0.0).astype(jnp.float32)
    hrows = lax.broadcasted_iota(jnp.int32, (HL, HD), 0)
    hcols = lax.broadcasted_iota(jnp.int32, (HL, HD), 1)
    bmat = jnp.where(hcols // DH == hrows, 1.0, 0.0).astype(jnp.float32)

    zsum = jnp.sum(z_ref[...], axis=1)

    out = jnp.dot(x_ref[...], w1_ref[...], preferred_element_type=jnp.float32)
    for c, (a_ref, w2_ref) in enumerate(((a0_ref, w2a_ref), (a1_ref, w2b_ref))):
        u = (jnp.dot(s0, a_ref[:, :HD], preferred_element_type=jnp.float32)
             + jnp.dot(s1, a_ref[:, HD:], preferred_element_type=jnp.float32))
        denom = lax.dot_general(zsum[c], bmat, (((0,), (0,)), ((), ())),
                                preferred_element_type=jnp.float32)
        msg = jnp.where(denom > 0, u / denom, 0.0)
        out = out + jnp.dot(msg, w2_ref[...], preferred_element_type=jnp.float32)
    o_ref[...] = out + b_ref[...]


def _out_call(xp, a0, a1, zparts, w1, w2a, w2b, b):
    blk = 1024
    grid = (NPAD // blk,)
    return pl.pallas_call(
        _out_body,
        grid=grid,
        in_specs=[
            pl.BlockSpec((blk, D), lambda i: (i, 0)),
            pl.BlockSpec((blk // 2, D), lambda i: (i, 0)),
            pl.BlockSpec((blk // 2, D), lambda i: (i, 0)),
            pl.BlockSpec((NC, NS, HL, blk), lambda i: (0, 0, 0, i)),
            pl.BlockSpec((D, D), lambda i: (0, 0)),
            pl.BlockSpec((HD, D), lambda i: (0, 0)),
            pl.BlockSpec((HD, D), lambda i: (0, 0)),
            pl.BlockSpec((1, D), lambda i: (0, 0)),
        ],
        out_specs=pl.BlockSpec((blk, D), lambda i: (i, 0)),
        out_shape=jax.ShapeDtypeStruct((NPAD, D), jnp.float32),
    )(xp, a0, a1, zparts, w1, w2a, w2b, b)


def kernel(x, edge_index, W_qkv, b_qkv, W_out, b_out):
    q_cols = np.concatenate([96 * h + np.arange(32) for h in range(H)])
    k_cols = np.concatenate([96 * h + 32 + np.arange(32) for h in range(H)])
    v_cols = np.concatenate([96 * h + 64 + np.arange(32) for h in range(H)])

    wq = W_qkv[:, q_cols] * COEF
    bq = (b_qkv[q_cols] * COEF).reshape(1, D)
    kv_cols = [np.concatenate([k_cols[HD * c:HD * (c + 1)],
                               v_cols[HD * c:HD * (c + 1)]]) for c in range(NC)]
    wkv = jnp.stack([W_qkv[:, kv_cols[c]] for c in range(NC)])
    bkv = jnp.stack([b_qkv[kv_cols[c]] for c in range(NC)]).reshape(NC, 1, 2 * HD)

    q, kv = _qkv_call(x, wq, bq, wkv, bkv)
    kv = kv.reshape(NC * N, 2 * HD)

    src = edge_index[0]
    dst = edge_index[1]
    acc, zparts = _sc_edge_call(q, kv, src, dst)
    zparts = zparts.reshape(NC, NS, HL, NPAD)

    xp = jnp.concatenate([x, jnp.zeros((NPAD - N, D), jnp.float32)], axis=0)
    out = _out_call(xp, acc[0], acc[1], zparts,
                    W_out[:D], W_out[D:D + HD], W_out[D + HD:], b_out.reshape(1, D))
    return out[:N]
